# baseline (speedup 1.0000x reference)
import functools
import jax
import jax.numpy as jnp
from jax import lax
from jax.experimental import pallas as pl
from jax.experimental.pallas import tpu as pltpu
from jax.experimental.pallas import tpu_sc as plsc

D_MODEL = 1024
NC, NS = 2, 16
NW = NC * NS
CH = 8
NBUF = 2


def _sc_body(x_hbm, p_hbm, o_hbm, buf, sem_in0, sem_in1, sem_out0, sem_out1):
    B = 4
    S = 8192
    rows_per_w = S // NW
    nchunks = rows_per_w // CH
    wid = lax.axis_index("s") * NC + lax.axis_index("c")
    seq0 = wid * rows_per_w
    sems_in = (sem_in0, sem_in1)
    sems_out = (sem_out0, sem_out1)

    def in_copies(c, par):
        base = seq0 + c * CH
        cps = [pltpu.make_async_copy(
            p_hbm.at[pl.ds(base, CH)], buf.at[par, 0], sems_in[par])]
        for b in range(B):
            cps.append(pltpu.make_async_copy(
                x_hbm.at[pl.ds(b * S + base, CH)], buf.at[par, 1 + b],
                sems_in[par]))
        return cps

    def out_copies(c, par):
        base = seq0 + c * CH
        return [pltpu.make_async_copy(
            buf.at[par, 1 + b], o_hbm.at[pl.ds(b * S + base, CH)],
            sems_out[par]) for b in range(B)]

    def compute(par):
        def jbody(j, _):
            col = j * 16
            for r in range(CH):
                p = buf[par, 0, r, pl.ds(col, 16)]
                for b in range(B):
                    buf[par, 1 + b, r, pl.ds(col, 16)] = (
                        buf[par, 1 + b, r, pl.ds(col, 16)] + p)
            return _
        lax.fori_loop(0, D_MODEL // 16, jbody, 0)

    def gbody(g, _):
        for par in range(NBUF):
            c = g * NBUF + par

            @pl.when(g > 0)
            def _drain():
                for cp in out_copies(c, par):
                    cp.wait()

            for cp in in_copies(c, par):
                cp.start()
        for par in range(NBUF):
            c = g * NBUF + par
            for cp in in_copies(c, par):
                cp.wait()
            compute(par)
            for cp in out_copies(c, par):
                cp.start()
        return _

    lax.fori_loop(0, nchunks // NBUF, gbody, 0)
    for par in range(NBUF):
        for cp in out_copies(nchunks - NBUF + par, par):
            cp.wait()


def kernel(x, pos_weight):
    B, S, D = x.shape
    x2 = x.reshape(B * S, D)
    mesh = plsc.VectorSubcoreMesh(core_axis_name="c", subcore_axis_name="s")
    run = functools.partial(
        pl.kernel,
        mesh=mesh,
        out_type=jax.ShapeDtypeStruct((B * S, D), jnp.float32),
        scratch_types=[
            pltpu.VMEM((NBUF, 5, CH, D), jnp.float32),
            pltpu.SemaphoreType.DMA,
            pltpu.SemaphoreType.DMA,
            pltpu.SemaphoreType.DMA,
            pltpu.SemaphoreType.DMA,
        ],
    )(_sc_body)
    out = run(x2, pos_weight[:S])
    return out.reshape(B, S, D)
